# baseline (speedup 1.0000x reference)
import functools

import jax
import jax.numpy as jnp
from jax import lax
from jax.experimental import pallas as pl
from jax.experimental.pallas import tpu as pltpu
from jax.experimental.pallas import tpu_sc as plsc

F = 128
CHUNK = 96
NBUF = 4
NW = 32

_TH = ((3.0, -3.0, 0.75), (0.0, 3.0, -1.5), (0.0, 0.0, 0.75))


def _round_up(x, m):
    return ((x + m - 1) // m) * m


@functools.lru_cache(maxsize=None)
def _make_prop(ep, npad):
    epw = ep // NW
    nchunks = epw // CHUNK
    ngroups = nchunks // NBUF
    rpt = npad // 16
    nzfull, nzrem = divmod(rpt, CHUNK)
    mesh = plsc.VectorSubcoreMesh(core_axis_name="c", subcore_axis_name="s")

    def body(g_hbm, src_hbm, dst_hbm, out0_hbm, out1_hbm,
             sv0, sv1, sv2, sv3, dv0, dv1, dv2, dv3,
             rb0, rb1, rb2, rb3, acc,
             gs0, gs1, gs2, gs3, ss0, ss1, ss2, ss3):
        cid = lax.axis_index("c")
        sid = lax.axis_index("s")
        svs = (sv0, sv1, sv2, sv3)
        dvs = (dv0, dv1, dv2, dv3)
        rbs = (rb0, rb1, rb2, rb3)
        gss = (gs0, gs1, gs2, gs3)
        sss = (ss0, ss1, ss2, ss3)
        zero16 = jnp.zeros((16,), jnp.float32)

        def fill_z(i, c):
            for j in range(F // 16):
                rb0[i, pl.ds(j * 16, 16)] = zero16
            return c
        lax.fori_loop(0, CHUNK, fill_z, 0)

        r0 = sid * rpt
        for k in range(nzfull):
            pltpu.sync_copy(rb0, acc.at[pl.ds(r0 + k * CHUNK, CHUNK)])
        if nzrem:
            pltpu.sync_copy(rb0.at[pl.ds(0, nzrem)],
                            acc.at[pl.ds(r0 + nzfull * CHUNK, nzrem)])
        plsc.subcore_barrier()

        base = (cid * 16 + sid) * epw

        def stage_fire(j, b):
            pltpu.sync_copy(src_hbm.at[pl.ds(base + j * CHUNK, CHUNK)], svs[b])
            pltpu.sync_copy(dst_hbm.at[pl.ds(base + j * CHUNK, CHUNK)], dvs[b])
            pltpu.async_copy(g_hbm.at[svs[b]], rbs[b], gss[b])

        def wait_gather(b):
            pltpu.make_async_copy(g_hbm.at[svs[b]], rbs[b], gss[b]).wait()

        def fire_scatter(b):
            pltpu.async_copy(rbs[b], acc.at[dvs[b]], sss[b], add=True)

        def wait_scatter(b):
            pltpu.make_async_copy(rbs[b], acc.at[dvs[b]], sss[b]).wait()

        for k in range(NBUF - 1):
            stage_fire(k, k)

        for k in range(NBUF):
            wait_gather(k)
            fire_scatter(k)
            if k > 0:
                wait_scatter(k - 1)
            stage_fire(k + NBUF - 1, (k - 1) % NBUF)

        def group(q, c):
            for k in range(NBUF):
                j = q * NBUF + k
                wait_gather(k)
                fire_scatter(k)
                wait_scatter((k - 1) % NBUF)
                stage_fire(j + NBUF - 1, (k - 1) % NBUF)
            return c
        lax.fori_loop(1, ngroups - 1, group, 0)

        for k in range(NBUF):
            wait_gather(k)
            fire_scatter(k)
            wait_scatter((k - 1) % NBUF)
            if k == 0:
                stage_fire(nchunks - 1, NBUF - 1)
        wait_scatter(NBUF - 1)

        plsc.subcore_barrier()

        @pl.when(cid == 0)
        def _():
            pltpu.sync_copy(acc.at[pl.ds(r0, rpt)], out0_hbm.at[pl.ds(r0, rpt)])

        @pl.when(cid == 1)
        def _():
            pltpu.sync_copy(acc.at[pl.ds(r0, rpt)], out1_hbm.at[pl.ds(r0, rpt)])

    return pl.kernel(
        body,
        out_type=[jax.ShapeDtypeStruct((npad, F), jnp.float32),
                  jax.ShapeDtypeStruct((npad, F), jnp.float32)],
        mesh=mesh,
        scratch_types=(
            [pltpu.VMEM((CHUNK,), jnp.int32)] * 8
            + [pltpu.VMEM((CHUNK, F), jnp.float32)] * 4
            + [pltpu.VMEM_SHARED((npad, F), jnp.float32)]
            + [pltpu.SemaphoreType.DMA] * 8
        ),
    )


@functools.lru_cache(maxsize=None)
def _make_deg(ep, npad):
    epw = ep // NW
    nchunks = epw // CHUNK
    rpt = npad // 16
    nzfull, nzrem = divmod(rpt, CHUNK)
    mesh = plsc.VectorSubcoreMesh(core_axis_name="c", subcore_axis_name="s")

    def body(dst_hbm, out0_hbm, out1_hbm, dv0, dv1, ones_v, acc, ss0, ss1):
        cid = lax.axis_index("c")
        sid = lax.axis_index("s")
        zero16 = jnp.zeros((16,), jnp.float32)
        one16 = jnp.ones((16,), jnp.float32)

        def fill_z(i, c):
            for j in range(F // 16):
                ones_v[i, pl.ds(j * 16, 16)] = zero16
            return c
        lax.fori_loop(0, CHUNK, fill_z, 0)

        r0 = sid * rpt
        for k in range(nzfull):
            pltpu.sync_copy(ones_v, acc.at[pl.ds(r0 + k * CHUNK, CHUNK)])
        if nzrem:
            pltpu.sync_copy(ones_v.at[pl.ds(0, nzrem)],
                            acc.at[pl.ds(r0 + nzfull * CHUNK, nzrem)])

        def fill_o(i, c):
            for j in range(F // 16):
                ones_v[i, pl.ds(j * 16, 16)] = one16
            return c
        lax.fori_loop(0, CHUNK, fill_o, 0)
        plsc.subcore_barrier()

        base = (cid * 16 + sid) * epw
        nsteps = nchunks // 2

        pltpu.sync_copy(dst_hbm.at[pl.ds(base, CHUNK)], dv0)

        def step(q, c):
            off = base + q * (2 * CHUNK)
            s0 = pltpu.async_copy(ones_v, acc.at[dv0], ss0, add=True)
            pltpu.sync_copy(dst_hbm.at[pl.ds(off + CHUNK, CHUNK)], dv1)
            s1 = pltpu.async_copy(ones_v, acc.at[dv1], ss1, add=True)
            s0.wait()
            pltpu.sync_copy(dst_hbm.at[pl.ds(off + 2 * CHUNK, CHUNK)], dv0)
            s1.wait()
            return c
        lax.fori_loop(0, nsteps - 1, step, 0)

        offl = base + (nsteps - 1) * (2 * CHUNK)
        s0 = pltpu.async_copy(ones_v, acc.at[dv0], ss0, add=True)
        pltpu.sync_copy(dst_hbm.at[pl.ds(offl + CHUNK, CHUNK)], dv1)
        s1 = pltpu.async_copy(ones_v, acc.at[dv1], ss1, add=True)
        s0.wait()
        s1.wait()

        plsc.subcore_barrier()

        @pl.when(cid == 0)
        def _():
            pltpu.sync_copy(acc.at[pl.ds(r0, rpt)], out0_hbm.at[pl.ds(r0, rpt)])

        @pl.when(cid == 1)
        def _():
            pltpu.sync_copy(acc.at[pl.ds(r0, rpt)], out1_hbm.at[pl.ds(r0, rpt)])

    return pl.kernel(
        body,
        out_type=[jax.ShapeDtypeStruct((npad, F), jnp.float32),
                  jax.ShapeDtypeStruct((npad, F), jnp.float32)],
        mesh=mesh,
        scratch_types=[
            pltpu.VMEM((CHUNK,), jnp.int32),
            pltpu.VMEM((CHUNK,), jnp.int32),
            pltpu.VMEM((CHUNK, F), jnp.float32),
            pltpu.VMEM_SHARED((npad, F), jnp.float32),
            pltpu.SemaphoreType.DMA,
            pltpu.SemaphoreType.DMA,
        ],
    )


def _mm_t(x, w):
    return lax.dot_general(x, w, (((1,), (1,)), ((), ())),
                           preferred_element_type=jnp.float32)


def _pre_body(nrows, bn, x_ref, w1_ref, b1_ref, w2_ref, b2_ref,
              dp0_ref, dp1_ref, h_ref, g_ref, di_ref):
    i = pl.program_id(0)
    x = x_ref[...]
    h1 = jax.nn.relu(_mm_t(x, w1_ref[...]) + b1_ref[...])
    h2 = jax.nn.relu(_mm_t(h1, w2_ref[...]) + b2_ref[...])
    deg = dp0_ref[...][:, 0:1] + dp1_ref[...][:, 0:1]
    dinv = lax.rsqrt(jnp.maximum(deg, 1.0))
    rows = lax.broadcasted_iota(jnp.int32, (bn, 1), 0) + i * bn
    mask = (rows < nrows).astype(jnp.float32)
    h2 = h2 * mask
    h_ref[...] = h2
    g_ref[...] = h2 * dinv
    di_ref[...] = jnp.broadcast_to(dinv, (bn, 16))


def _mid_body(nrows, bn, h_ref, p0_ref, p1_ref, di_ref, c_ref, g_ref):
    i = pl.program_id(0)
    dinv = di_ref[...][:, 0:1]
    cur = h_ref[...] - (p0_ref[...] + p1_ref[...]) * dinv
    rows = lax.broadcasted_iota(jnp.int32, (bn, 1), 0) + i * bn
    mask = (rows < nrows).astype(jnp.float32)
    cur = cur * mask
    c_ref[...] = cur
    g_ref[...] = cur * dinv


def _post_body(h_ref, c1_ref, p0_ref, p1_ref, di_ref, w3_ref, b3_ref,
               w4a_ref, w4b_ref, b4_ref, hl_ref, hh_ref):
    dinv = di_ref[...][:, 0:1]
    h = h_ref[...]
    c1 = c1_ref[...]
    c2 = c1 - (p0_ref[...] + p1_ref[...]) * dinv
    o0 = _TH[0][0] * h + _TH[0][1] * c1 + _TH[0][2] * c2
    o1 = _TH[1][1] * c1 + _TH[1][2] * c2
    o2 = _TH[2][2] * c2
    hl_ref[...] = jax.nn.relu(_mm_t(o0, w3_ref[...]) + b3_ref[...])
    hh_ref[...] = jax.nn.relu(_mm_t(o1, w4a_ref[...]) + _mm_t(o2, w4b_ref[...])
                              + b4_ref[...])


def kernel(in_feat, edge_index, W1, b1, W2, b2, W3, b3, W4, b4):
    n, f = in_feat.shape
    e = edge_index.shape[1]
    npad = _round_up(n + 16, 128)
    ep = _round_up(e, NW * CHUNK * NBUF)

    src = edge_index[0].astype(jnp.int32)
    dst = edge_index[1].astype(jnp.int32)
    sent = jnp.full((ep - e,), n, jnp.int32)
    srcp = jnp.concatenate([src, sent])
    dstp = jnp.concatenate([dst, sent])
    xpad = jnp.pad(in_feat, ((0, npad - n), (0, 0)))
    b1r, b2r, b3r, b4r = (x.reshape(1, f) for x in (b1, b2, b3, b4))
    W4a, W4b = W4[:, :f], W4[:, f:]

    prop = _make_prop(ep, npad)
    dp0, dp1 = _make_deg(ep, npad)(dstp)

    bn = npad // 4
    wspec = pl.BlockSpec((f, f), lambda i: (0, 0))
    bspec = pl.BlockSpec((1, f), lambda i: (0, 0))
    rspec = pl.BlockSpec((bn, f), lambda i: (i, 0))
    dspec = pl.BlockSpec((bn, 16), lambda i: (i, 0))
    rshape = jax.ShapeDtypeStruct((npad, f), jnp.float32)

    h, g1, dinv16 = pl.pallas_call(
        functools.partial(_pre_body, n, bn),
        grid=(npad // bn,),
        in_specs=[rspec, wspec, bspec, wspec, bspec, rspec, rspec],
        out_specs=[rspec, rspec, dspec],
        out_shape=[rshape, rshape,
                   jax.ShapeDtypeStruct((npad, 16), jnp.float32)],
    )(xpad, W1, b1r, W2, b2r, dp0, dp1)

    p10, p11 = prop(g1, srcp, dstp)

    cur1, g2 = pl.pallas_call(
        functools.partial(_mid_body, n, bn),
        grid=(npad // bn,),
        in_specs=[rspec, rspec, rspec, dspec],
        out_specs=[rspec, rspec],
        out_shape=[rshape, rshape],
    )(h, p10, p11, dinv16)

    p20, p21 = prop(g2, srcp, dstp)

    hl, hh = pl.pallas_call(
        _post_body,
        grid=(npad // bn,),
        in_specs=[rspec, rspec, rspec, rspec, dspec,
                  wspec, bspec, wspec, wspec, bspec],
        out_specs=[rspec, rspec],
        out_shape=[rshape, rshape],
    )(h, cur1, p20, p21, dinv16, W3, b3r, W4a, W4b, b4r)

    return hl[:n], hh[:n]


# speedup vs baseline: 1.2998x; 1.2998x over previous
import functools

import jax
import jax.numpy as jnp
from jax import lax
from jax.experimental import pallas as pl
from jax.experimental.pallas import tpu as pltpu
from jax.experimental.pallas import tpu_sc as plsc

F = 128
CHUNK = 128
NW = 32

_TH = ((3.0, -3.0, 0.75), (0.0, 3.0, -1.5), (0.0, 0.0, 0.75))


def _round_up(x, m):
    return ((x + m - 1) // m) * m


@functools.lru_cache(maxsize=None)
def _make_prop(ep, npad, cw0):
    nchunks = ep // (NW * CHUNK)
    cw1 = 2 * nchunks - cw0
    assert cw0 % 2 == 0 and cw1 % 2 == 0 and cw0 >= 4 and cw1 >= 4
    rpt = npad // 16
    nzfull, nzrem = divmod(rpt, CHUNK)
    mesh = plsc.VectorSubcoreMesh(core_axis_name="c", subcore_axis_name="s")

    def body(g_hbm, src_hbm, dst_hbm, out0_hbm, out1_hbm,
             sv0, sv1, dv0, dv1, rows0, rows1, acc, gs0, gs1, ss0, ss1):
        cid = lax.axis_index("c")
        sid = lax.axis_index("s")
        zero16 = jnp.zeros((16,), jnp.float32)

        def fill_z(i, c):
            for j in range(F // 16):
                rows0[i, pl.ds(j * 16, 16)] = zero16
            return c
        lax.fori_loop(0, CHUNK, fill_z, 0)

        r0 = sid * rpt
        for k in range(nzfull):
            pltpu.sync_copy(rows0, acc.at[pl.ds(r0 + k * CHUNK, CHUNK)])
        if nzrem:
            pltpu.sync_copy(rows0.at[pl.ds(0, nzrem)],
                            acc.at[pl.ds(r0 + nzfull * CHUNK, nzrem)])
        plsc.subcore_barrier()

        def run(base, nsteps):
            pltpu.sync_copy(src_hbm.at[pl.ds(base, CHUNK)], sv0)
            pltpu.sync_copy(dst_hbm.at[pl.ds(base, CHUNK)], dv0)
            pltpu.async_copy(g_hbm.at[sv0], rows0, gs0)

            def step(q, c):
                off = base + q * (2 * CHUNK)
                pltpu.sync_copy(src_hbm.at[pl.ds(off + CHUNK, CHUNK)], sv1)
                pltpu.sync_copy(dst_hbm.at[pl.ds(off + CHUNK, CHUNK)], dv1)
                pltpu.async_copy(g_hbm.at[sv1], rows1, gs1)
                pltpu.make_async_copy(g_hbm.at[sv0], rows0, gs0).wait()
                pltpu.async_copy(rows0, acc.at[dv0], ss0, add=True).wait()

                off2 = off + 2 * CHUNK
                pltpu.sync_copy(src_hbm.at[pl.ds(off2, CHUNK)], sv0)
                pltpu.sync_copy(dst_hbm.at[pl.ds(off2, CHUNK)], dv0)
                pltpu.async_copy(g_hbm.at[sv0], rows0, gs0)

                pltpu.make_async_copy(g_hbm.at[sv1], rows1, gs1).wait()
                pltpu.async_copy(rows1, acc.at[dv1], ss1, add=True).wait()
                return c
            lax.fori_loop(0, nsteps - 1, step, 0)

            offl = base + (nsteps - 1) * (2 * CHUNK)
            pltpu.sync_copy(src_hbm.at[pl.ds(offl + CHUNK, CHUNK)], sv1)
            pltpu.sync_copy(dst_hbm.at[pl.ds(offl + CHUNK, CHUNK)], dv1)
            pltpu.async_copy(g_hbm.at[sv1], rows1, gs1)
            pltpu.make_async_copy(g_hbm.at[sv0], rows0, gs0).wait()
            pltpu.async_copy(rows0, acc.at[dv0], ss0, add=True).wait()
            pltpu.make_async_copy(g_hbm.at[sv1], rows1, gs1).wait()
            pltpu.async_copy(rows1, acc.at[dv1], ss1, add=True).wait()

        @pl.when(cid == 0)
        def _():
            run(sid * (cw0 * CHUNK), cw0 // 2)

        @pl.when(cid == 1)
        def _():
            run((16 * cw0 + sid * cw1) * CHUNK, cw1 // 2)

        plsc.subcore_barrier()

        @pl.when(cid == 0)
        def _():
            pltpu.sync_copy(acc.at[pl.ds(r0, rpt)], out0_hbm.at[pl.ds(r0, rpt)])

        @pl.when(cid == 1)
        def _():
            pltpu.sync_copy(acc.at[pl.ds(r0, rpt)], out1_hbm.at[pl.ds(r0, rpt)])

    return pl.kernel(
        body,
        out_type=[jax.ShapeDtypeStruct((npad, F), jnp.float32),
                  jax.ShapeDtypeStruct((npad, F), jnp.float32)],
        mesh=mesh,
        scratch_types=(
            [pltpu.VMEM((CHUNK,), jnp.int32)] * 4
            + [pltpu.VMEM((CHUNK, F), jnp.float32)] * 2
            + [pltpu.VMEM_SHARED((npad, F), jnp.float32)]
            + [pltpu.SemaphoreType.DMA] * 4
        ),
    )


@functools.lru_cache(maxsize=None)
def _make_deg(ep, npad):
    epw = ep // NW
    nchunks = epw // CHUNK
    rpt = npad // 16
    nzfull, nzrem = divmod(rpt, CHUNK)
    mesh = plsc.VectorSubcoreMesh(core_axis_name="c", subcore_axis_name="s")

    def body(dst_hbm, out0_hbm, out1_hbm, dv0, dv1, ones_v, acc, ss0, ss1):
        cid = lax.axis_index("c")
        sid = lax.axis_index("s")
        zero16 = jnp.zeros((16,), jnp.float32)
        one16 = jnp.ones((16,), jnp.float32)

        def fill_z(i, c):
            for j in range(F // 16):
                ones_v[i, pl.ds(j * 16, 16)] = zero16
            return c
        lax.fori_loop(0, CHUNK, fill_z, 0)

        r0 = sid * rpt
        for k in range(nzfull):
            pltpu.sync_copy(ones_v, acc.at[pl.ds(r0 + k * CHUNK, CHUNK)])
        if nzrem:
            pltpu.sync_copy(ones_v.at[pl.ds(0, nzrem)],
                            acc.at[pl.ds(r0 + nzfull * CHUNK, nzrem)])

        def fill_o(i, c):
            for j in range(F // 16):
                ones_v[i, pl.ds(j * 16, 16)] = one16
            return c
        lax.fori_loop(0, CHUNK, fill_o, 0)
        plsc.subcore_barrier()

        base = (cid * 16 + sid) * epw
        nsteps = nchunks // 2

        pltpu.sync_copy(dst_hbm.at[pl.ds(base, CHUNK)], dv0)

        def step(q, c):
            off = base + q * (2 * CHUNK)
            s0 = pltpu.async_copy(ones_v, acc.at[dv0], ss0, add=True)
            pltpu.sync_copy(dst_hbm.at[pl.ds(off + CHUNK, CHUNK)], dv1)
            s1 = pltpu.async_copy(ones_v, acc.at[dv1], ss1, add=True)
            s0.wait()
            pltpu.sync_copy(dst_hbm.at[pl.ds(off + 2 * CHUNK, CHUNK)], dv0)
            s1.wait()
            return c
        lax.fori_loop(0, nsteps - 1, step, 0)

        offl = base + (nsteps - 1) * (2 * CHUNK)
        s0 = pltpu.async_copy(ones_v, acc.at[dv0], ss0, add=True)
        pltpu.sync_copy(dst_hbm.at[pl.ds(offl + CHUNK, CHUNK)], dv1)
        s1 = pltpu.async_copy(ones_v, acc.at[dv1], ss1, add=True)
        s0.wait()
        s1.wait()

        plsc.subcore_barrier()

        @pl.when(cid == 0)
        def _():
            pltpu.sync_copy(acc.at[pl.ds(r0, rpt)], out0_hbm.at[pl.ds(r0, rpt)])

        @pl.when(cid == 1)
        def _():
            pltpu.sync_copy(acc.at[pl.ds(r0, rpt)], out1_hbm.at[pl.ds(r0, rpt)])

    return pl.kernel(
        body,
        out_type=[jax.ShapeDtypeStruct((npad, F), jnp.float32),
                  jax.ShapeDtypeStruct((npad, F), jnp.float32)],
        mesh=mesh,
        scratch_types=[
            pltpu.VMEM((CHUNK,), jnp.int32),
            pltpu.VMEM((CHUNK,), jnp.int32),
            pltpu.VMEM((CHUNK, F), jnp.float32),
            pltpu.VMEM_SHARED((npad, F), jnp.float32),
            pltpu.SemaphoreType.DMA,
            pltpu.SemaphoreType.DMA,
        ],
    )


def _mm_t(x, w):
    return lax.dot_general(x, w, (((1,), (1,)), ((), ())),
                           preferred_element_type=jnp.float32)


def _pre_body(nrows, bn, x_ref, w1_ref, b1_ref, w2_ref, b2_ref,
              dp0_ref, dp1_ref, h_ref, g_ref, di_ref):
    i = pl.program_id(0)
    x = x_ref[...]
    h1 = jax.nn.relu(_mm_t(x, w1_ref[...]) + b1_ref[...])
    h2 = jax.nn.relu(_mm_t(h1, w2_ref[...]) + b2_ref[...])
    deg = dp0_ref[...][:, 0:1] + dp1_ref[...][:, 0:1]
    dinv = lax.rsqrt(jnp.maximum(deg, 1.0))
    rows = lax.broadcasted_iota(jnp.int32, (bn, 1), 0) + i * bn
    mask = (rows < nrows).astype(jnp.float32)
    h2 = h2 * mask
    h_ref[...] = h2
    g_ref[...] = h2 * dinv
    di_ref[...] = jnp.broadcast_to(dinv, (bn, 16))


def _mid_body(nrows, bn, h_ref, p0_ref, p1_ref, di_ref, c_ref, g_ref):
    i = pl.program_id(0)
    dinv = di_ref[...][:, 0:1]
    cur = h_ref[...] - (p0_ref[...] + p1_ref[...]) * dinv
    rows = lax.broadcasted_iota(jnp.int32, (bn, 1), 0) + i * bn
    mask = (rows < nrows).astype(jnp.float32)
    cur = cur * mask
    c_ref[...] = cur
    g_ref[...] = cur * dinv


def _post_body(h_ref, c1_ref, p0_ref, p1_ref, di_ref, w3_ref, b3_ref,
               w4a_ref, w4b_ref, b4_ref, hl_ref, hh_ref):
    dinv = di_ref[...][:, 0:1]
    h = h_ref[...]
    c1 = c1_ref[...]
    c2 = c1 - (p0_ref[...] + p1_ref[...]) * dinv
    o0 = _TH[0][0] * h + _TH[0][1] * c1 + _TH[0][2] * c2
    o1 = _TH[1][1] * c1 + _TH[1][2] * c2
    o2 = _TH[2][2] * c2
    hl_ref[...] = jax.nn.relu(_mm_t(o0, w3_ref[...]) + b3_ref[...])
    hh_ref[...] = jax.nn.relu(_mm_t(o1, w4a_ref[...]) + _mm_t(o2, w4b_ref[...])
                              + b4_ref[...])


def kernel(in_feat, edge_index, W1, b1, W2, b2, W3, b3, W4, b4):
    n, f = in_feat.shape
    e = edge_index.shape[1]
    npad = _round_up(n + 16, 128)
    ep = _round_up(e, NW * CHUNK * 4)

    src = edge_index[0].astype(jnp.int32)
    dst = edge_index[1].astype(jnp.int32)
    sent = jnp.full((ep - e,), n, jnp.int32)
    srcp = jnp.concatenate([src, sent])
    dstp = jnp.concatenate([dst, sent])
    xpad = jnp.pad(in_feat, ((0, npad - n), (0, 0)))
    b1r, b2r, b3r, b4r = (x.reshape(1, f) for x in (b1, b2, b3, b4))
    W4a, W4b = W4[:, :f], W4[:, f:]

    nchunks_even = ep // (NW * CHUNK)
    cw0 = _round_up(int(nchunks_even * 2 * 0.40), 2)
    prop = _make_prop(ep, npad, cw0)
    dp0, dp1 = _make_deg(ep, npad)(dstp)

    bn = npad // 4
    wspec = pl.BlockSpec((f, f), lambda i: (0, 0))
    bspec = pl.BlockSpec((1, f), lambda i: (0, 0))
    rspec = pl.BlockSpec((bn, f), lambda i: (i, 0))
    dspec = pl.BlockSpec((bn, 16), lambda i: (i, 0))
    rshape = jax.ShapeDtypeStruct((npad, f), jnp.float32)

    h, g1, dinv16 = pl.pallas_call(
        functools.partial(_pre_body, n, bn),
        grid=(npad // bn,),
        in_specs=[rspec, wspec, bspec, wspec, bspec, rspec, rspec],
        out_specs=[rspec, rspec, dspec],
        out_shape=[rshape, rshape,
                   jax.ShapeDtypeStruct((npad, 16), jnp.float32)],
    )(xpad, W1, b1r, W2, b2r, dp0, dp1)

    p10, p11 = prop(g1, srcp, dstp)

    cur1, g2 = pl.pallas_call(
        functools.partial(_mid_body, n, bn),
        grid=(npad // bn,),
        in_specs=[rspec, rspec, rspec, dspec],
        out_specs=[rspec, rspec],
        out_shape=[rshape, rshape],
    )(h, p10, p11, dinv16)

    p20, p21 = prop(g2, srcp, dstp)

    hl, hh = pl.pallas_call(
        _post_body,
        grid=(npad // bn,),
        in_specs=[rspec, rspec, rspec, rspec, dspec,
                  wspec, bspec, wspec, wspec, bspec],
        out_specs=[rspec, rspec],
        out_shape=[rshape, rshape],
    )(h, cur1, p20, p21, dinv16, W3, b3r, W4a, W4b, b4r)

    return hl[:n], hh[:n]


# speedup vs baseline: 1.3985x; 1.0759x over previous
import functools

import jax
import jax.numpy as jnp
from jax import lax
from jax.experimental import pallas as pl
from jax.experimental.pallas import tpu as pltpu
from jax.experimental.pallas import tpu_sc as plsc

F = 128
CHUNK = 128
NW = 32

_TH = ((3.0, -3.0, 0.75), (0.0, 3.0, -1.5), (0.0, 0.0, 0.75))


def _round_up(x, m):
    return ((x + m - 1) // m) * m


@functools.lru_cache(maxsize=None)
def _make_prop(ep, npad, cw0):
    nchunks = ep // (NW * CHUNK)
    cw1 = 2 * nchunks - cw0
    assert cw0 % 2 == 0 and cw1 % 2 == 0 and cw0 >= 4 and cw1 >= 4
    rpt = npad // 16
    nzfull, nzrem = divmod(rpt, CHUNK)
    mesh = plsc.VectorSubcoreMesh(core_axis_name="c", subcore_axis_name="s")

    def body(g_hbm, src_hbm, dst_hbm, out0_hbm, out1_hbm,
             sv0, sv1, dv0, dv1, rows0, rows1, acc, gs0, gs1, ss0, ss1):
        cid = lax.axis_index("c")
        sid = lax.axis_index("s")
        zero16 = jnp.zeros((16,), jnp.float32)

        def fill_z(i, c):
            for j in range(F // 16):
                rows0[i, pl.ds(j * 16, 16)] = zero16
            return c
        lax.fori_loop(0, CHUNK, fill_z, 0)

        r0 = sid * rpt
        for k in range(nzfull):
            pltpu.sync_copy(rows0, acc.at[pl.ds(r0 + k * CHUNK, CHUNK)])
        if nzrem:
            pltpu.sync_copy(rows0.at[pl.ds(0, nzrem)],
                            acc.at[pl.ds(r0 + nzfull * CHUNK, nzrem)])
        plsc.subcore_barrier()

        def run(base, nsteps):
            pltpu.sync_copy(src_hbm.at[pl.ds(base, CHUNK)], sv0)
            pltpu.sync_copy(dst_hbm.at[pl.ds(base, CHUNK)], dv0)
            pltpu.async_copy(g_hbm.at[sv0], rows0, gs0)

            def step(q, c):
                off = base + q * (2 * CHUNK)
                pltpu.sync_copy(src_hbm.at[pl.ds(off + CHUNK, CHUNK)], sv1)
                pltpu.sync_copy(dst_hbm.at[pl.ds(off + CHUNK, CHUNK)], dv1)
                pltpu.async_copy(g_hbm.at[sv1], rows1, gs1)
                pltpu.make_async_copy(g_hbm.at[sv0], rows0, gs0).wait()
                pltpu.async_copy(rows0, acc.at[dv0], ss0, add=True).wait()

                off2 = off + 2 * CHUNK
                pltpu.sync_copy(src_hbm.at[pl.ds(off2, CHUNK)], sv0)
                pltpu.sync_copy(dst_hbm.at[pl.ds(off2, CHUNK)], dv0)
                pltpu.async_copy(g_hbm.at[sv0], rows0, gs0)

                pltpu.make_async_copy(g_hbm.at[sv1], rows1, gs1).wait()
                pltpu.async_copy(rows1, acc.at[dv1], ss1, add=True).wait()
                return c
            lax.fori_loop(0, nsteps - 1, step, 0)

            offl = base + (nsteps - 1) * (2 * CHUNK)
            pltpu.sync_copy(src_hbm.at[pl.ds(offl + CHUNK, CHUNK)], sv1)
            pltpu.sync_copy(dst_hbm.at[pl.ds(offl + CHUNK, CHUNK)], dv1)
            pltpu.async_copy(g_hbm.at[sv1], rows1, gs1)
            pltpu.make_async_copy(g_hbm.at[sv0], rows0, gs0).wait()
            pltpu.async_copy(rows0, acc.at[dv0], ss0, add=True).wait()
            pltpu.make_async_copy(g_hbm.at[sv1], rows1, gs1).wait()
            pltpu.async_copy(rows1, acc.at[dv1], ss1, add=True).wait()

        @pl.when(cid == 0)
        def _():
            run(sid * (cw0 * CHUNK), cw0 // 2)

        @pl.when(cid == 1)
        def _():
            run((16 * cw0 + sid * cw1) * CHUNK, cw1 // 2)

        plsc.subcore_barrier()

        @pl.when(cid == 0)
        def _():
            pltpu.sync_copy(acc.at[pl.ds(r0, rpt)], out0_hbm.at[pl.ds(r0, rpt)])

        @pl.when(cid == 1)
        def _():
            pltpu.sync_copy(acc.at[pl.ds(r0, rpt)], out1_hbm.at[pl.ds(r0, rpt)])

    return pl.kernel(
        body,
        out_type=[jax.ShapeDtypeStruct((npad, F), jnp.float32),
                  jax.ShapeDtypeStruct((npad, F), jnp.float32)],
        mesh=mesh,
        scratch_types=(
            [pltpu.VMEM((CHUNK,), jnp.int32)] * 4
            + [pltpu.VMEM((CHUNK, F), jnp.float32)] * 2
            + [pltpu.VMEM_SHARED((npad, F), jnp.float32)]
            + [pltpu.SemaphoreType.DMA] * 4
        ),
    )


@functools.lru_cache(maxsize=None)
def _make_deg(ep, npad):
    epw = ep // NW
    nchunks = epw // CHUNK
    rpt = npad // 16
    nzfull, nzrem = divmod(rpt, CHUNK)
    mesh = plsc.VectorSubcoreMesh(core_axis_name="c", subcore_axis_name="s")

    def body(dst_hbm, out0_hbm, out1_hbm, dv0, dv1, ones_v, acc, ss0, ss1):
        cid = lax.axis_index("c")
        sid = lax.axis_index("s")
        zero16 = jnp.zeros((16,), jnp.float32)
        one16 = jnp.ones((16,), jnp.float32)

        def fill_z(i, c):
            for j in range(F // 16):
                ones_v[i, pl.ds(j * 16, 16)] = zero16
            return c
        lax.fori_loop(0, CHUNK, fill_z, 0)

        r0 = sid * rpt
        for k in range(nzfull):
            pltpu.sync_copy(ones_v, acc.at[pl.ds(r0 + k * CHUNK, CHUNK)])
        if nzrem:
            pltpu.sync_copy(ones_v.at[pl.ds(0, nzrem)],
                            acc.at[pl.ds(r0 + nzfull * CHUNK, nzrem)])

        def fill_o(i, c):
            for j in range(F // 16):
                ones_v[i, pl.ds(j * 16, 16)] = one16
            return c
        lax.fori_loop(0, CHUNK, fill_o, 0)
        plsc.subcore_barrier()

        base = (cid * 16 + sid) * epw
        nsteps = nchunks // 2

        pltpu.sync_copy(dst_hbm.at[pl.ds(base, CHUNK)], dv0)

        def step(q, c):
            off = base + q * (2 * CHUNK)
            s0 = pltpu.async_copy(ones_v, acc.at[dv0], ss0, add=True)
            pltpu.sync_copy(dst_hbm.at[pl.ds(off + CHUNK, CHUNK)], dv1)
            s1 = pltpu.async_copy(ones_v, acc.at[dv1], ss1, add=True)
            s0.wait()
            pltpu.sync_copy(dst_hbm.at[pl.ds(off + 2 * CHUNK, CHUNK)], dv0)
            s1.wait()
            return c
        lax.fori_loop(0, nsteps - 1, step, 0)

        offl = base + (nsteps - 1) * (2 * CHUNK)
        s0 = pltpu.async_copy(ones_v, acc.at[dv0], ss0, add=True)
        pltpu.sync_copy(dst_hbm.at[pl.ds(offl + CHUNK, CHUNK)], dv1)
        s1 = pltpu.async_copy(ones_v, acc.at[dv1], ss1, add=True)
        s0.wait()
        s1.wait()

        plsc.subcore_barrier()

        @pl.when(cid == 0)
        def _():
            pltpu.sync_copy(acc.at[pl.ds(r0, rpt)], out0_hbm.at[pl.ds(r0, rpt)])

        @pl.when(cid == 1)
        def _():
            pltpu.sync_copy(acc.at[pl.ds(r0, rpt)], out1_hbm.at[pl.ds(r0, rpt)])

    return pl.kernel(
        body,
        out_type=[jax.ShapeDtypeStruct((npad, F), jnp.float32),
                  jax.ShapeDtypeStruct((npad, F), jnp.float32)],
        mesh=mesh,
        scratch_types=[
            pltpu.VMEM((CHUNK,), jnp.int32),
            pltpu.VMEM((CHUNK,), jnp.int32),
            pltpu.VMEM((CHUNK, F), jnp.float32),
            pltpu.VMEM_SHARED((npad, F), jnp.float32),
            pltpu.SemaphoreType.DMA,
            pltpu.SemaphoreType.DMA,
        ],
    )


def _mm_t(x, w):
    return lax.dot_general(x, w, (((1,), (1,)), ((), ())),
                           preferred_element_type=jnp.float32)


def _pre_body(nrows, bn, x_ref, w1_ref, b1_ref, w2_ref, b2_ref,
              dp0_ref, dp1_ref, h_ref, g_ref, di_ref):
    i = pl.program_id(0)
    x = x_ref[...]
    h1 = jax.nn.relu(_mm_t(x, w1_ref[...]) + b1_ref[...])
    h2 = jax.nn.relu(_mm_t(h1, w2_ref[...]) + b2_ref[...])
    deg = dp0_ref[...][:, 0:1] + dp1_ref[...][:, 0:1]
    dinv = lax.rsqrt(jnp.maximum(deg, 1.0))
    rows = lax.broadcasted_iota(jnp.int32, (bn, 1), 0) + i * bn
    mask = (rows < nrows).astype(jnp.float32)
    h2 = h2 * mask
    h_ref[...] = h2
    g_ref[...] = h2 * dinv
    di_ref[...] = jnp.broadcast_to(dinv, (bn, 16))


def _mid_body(nrows, bn, h_ref, p0_ref, p1_ref, di_ref, c_ref, g_ref):
    i = pl.program_id(0)
    dinv = di_ref[...][:, 0:1]
    cur = h_ref[...] - (p0_ref[...] + p1_ref[...]) * dinv
    rows = lax.broadcasted_iota(jnp.int32, (bn, 1), 0) + i * bn
    mask = (rows < nrows).astype(jnp.float32)
    cur = cur * mask
    c_ref[...] = cur
    g_ref[...] = cur * dinv


def _post_body(h_ref, c1_ref, p0_ref, p1_ref, di_ref, w3_ref, b3_ref,
               w4a_ref, w4b_ref, b4_ref, hl_ref, hh_ref):
    dinv = di_ref[...][:, 0:1]
    h = h_ref[...]
    c1 = c1_ref[...]
    c2 = c1 - (p0_ref[...] + p1_ref[...]) * dinv
    o0 = _TH[0][0] * h + _TH[0][1] * c1 + _TH[0][2] * c2
    o1 = _TH[1][1] * c1 + _TH[1][2] * c2
    o2 = _TH[2][2] * c2
    hl_ref[...] = jax.nn.relu(_mm_t(o0, w3_ref[...]) + b3_ref[...])
    hh_ref[...] = jax.nn.relu(_mm_t(o1, w4a_ref[...]) + _mm_t(o2, w4b_ref[...])
                              + b4_ref[...])


def kernel(in_feat, edge_index, W1, b1, W2, b2, W3, b3, W4, b4):
    n, f = in_feat.shape
    e = edge_index.shape[1]
    npad = _round_up(n + 16, 128)
    ep = _round_up(e, NW * CHUNK * 4)

    src = edge_index[0].astype(jnp.int32)
    dst = edge_index[1].astype(jnp.int32)
    sent = jnp.full((ep - e,), n, jnp.int32)
    srcp = jnp.concatenate([src, sent])
    dstp = jnp.concatenate([dst, sent])
    xpad = jnp.pad(in_feat, ((0, npad - n), (0, 0)))
    b1r, b2r, b3r, b4r = (x.reshape(1, f) for x in (b1, b2, b3, b4))
    W4a, W4b = W4[:, :f], W4[:, f:]

    nchunks_even = ep // (NW * CHUNK)
    cw0 = _round_up(int(nchunks_even * 2 * 0.60), 2)
    prop = _make_prop(ep, npad, cw0)
    dp0, dp1 = _make_deg(ep, npad)(dstp)

    bn = npad // 4
    wspec = pl.BlockSpec((f, f), lambda i: (0, 0))
    bspec = pl.BlockSpec((1, f), lambda i: (0, 0))
    rspec = pl.BlockSpec((bn, f), lambda i: (i, 0))
    dspec = pl.BlockSpec((bn, 16), lambda i: (i, 0))
    rshape = jax.ShapeDtypeStruct((npad, f), jnp.float32)

    h, g1, dinv16 = pl.pallas_call(
        functools.partial(_pre_body, n, bn),
        grid=(npad // bn,),
        in_specs=[rspec, wspec, bspec, wspec, bspec, rspec, rspec],
        out_specs=[rspec, rspec, dspec],
        out_shape=[rshape, rshape,
                   jax.ShapeDtypeStruct((npad, 16), jnp.float32)],
    )(xpad, W1, b1r, W2, b2r, dp0, dp1)

    p10, p11 = prop(g1, srcp, dstp)

    cur1, g2 = pl.pallas_call(
        functools.partial(_mid_body, n, bn),
        grid=(npad // bn,),
        in_specs=[rspec, rspec, rspec, dspec],
        out_specs=[rspec, rspec],
        out_shape=[rshape, rshape],
    )(h, p10, p11, dinv16)

    p20, p21 = prop(g2, srcp, dstp)

    hl, hh = pl.pallas_call(
        _post_body,
        grid=(npad // bn,),
        in_specs=[rspec, rspec, rspec, rspec, dspec,
                  wspec, bspec, wspec, wspec, bspec],
        out_specs=[rspec, rspec],
        out_shape=[rshape, rshape],
    )(h, cur1, p20, p21, dinv16, W3, b3r, W4a, W4b, b4r)

    return hl[:n], hh[:n]


# speedup vs baseline: 1.4260x; 1.0197x over previous
import functools

import jax
import jax.numpy as jnp
from jax import lax
from jax.experimental import pallas as pl
from jax.experimental.pallas import tpu as pltpu
from jax.experimental.pallas import tpu_sc as plsc

F = 128
CHUNK = 128
NW = 32

_TH = ((3.0, -3.0, 0.75), (0.0, 3.0, -1.5), (0.0, 0.0, 0.75))


def _round_up(x, m):
    return ((x + m - 1) // m) * m


@functools.lru_cache(maxsize=None)
def _make_prop(ep, npad, cw0):
    nchunks = ep // (NW * CHUNK)
    cw1 = 2 * nchunks - cw0
    assert cw0 % 2 == 0 and cw1 % 2 == 0 and cw0 >= 4 and cw1 >= 4
    rpt = npad // 16
    nzfull, nzrem = divmod(rpt, CHUNK)
    mesh = plsc.VectorSubcoreMesh(core_axis_name="c", subcore_axis_name="s")

    def body(g_hbm, src_hbm, dst_hbm, out0_hbm, out1_hbm,
             sv0, sv1, dv0, dv1, rows0, rows1, acc, gs0, gs1, ss0, ss1):
        cid = lax.axis_index("c")
        sid = lax.axis_index("s")
        zero16 = jnp.zeros((16,), jnp.float32)

        def fill_z(i, c):
            for j in range(F // 16):
                rows0[i, pl.ds(j * 16, 16)] = zero16
            return c
        lax.fori_loop(0, CHUNK, fill_z, 0)

        r0 = sid * rpt
        for k in range(nzfull):
            pltpu.sync_copy(rows0, acc.at[pl.ds(r0 + k * CHUNK, CHUNK)])
        if nzrem:
            pltpu.sync_copy(rows0.at[pl.ds(0, nzrem)],
                            acc.at[pl.ds(r0 + nzfull * CHUNK, nzrem)])
        plsc.subcore_barrier()

        def run(base, nsteps):
            pltpu.sync_copy(src_hbm.at[pl.ds(base, CHUNK)], sv0)
            pltpu.sync_copy(dst_hbm.at[pl.ds(base, CHUNK)], dv0)
            pltpu.async_copy(g_hbm.at[sv0], rows0, gs0)

            def step(q, c):
                off = base + q * (2 * CHUNK)
                pltpu.sync_copy(src_hbm.at[pl.ds(off + CHUNK, CHUNK)], sv1)
                pltpu.sync_copy(dst_hbm.at[pl.ds(off + CHUNK, CHUNK)], dv1)
                pltpu.async_copy(g_hbm.at[sv1], rows1, gs1)
                pltpu.make_async_copy(g_hbm.at[sv0], rows0, gs0).wait()
                pltpu.async_copy(rows0, acc.at[dv0], ss0, add=True).wait()

                off2 = off + 2 * CHUNK
                pltpu.sync_copy(src_hbm.at[pl.ds(off2, CHUNK)], sv0)
                pltpu.sync_copy(dst_hbm.at[pl.ds(off2, CHUNK)], dv0)
                pltpu.async_copy(g_hbm.at[sv0], rows0, gs0)

                pltpu.make_async_copy(g_hbm.at[sv1], rows1, gs1).wait()
                pltpu.async_copy(rows1, acc.at[dv1], ss1, add=True).wait()
                return c
            lax.fori_loop(0, nsteps - 1, step, 0)

            offl = base + (nsteps - 1) * (2 * CHUNK)
            pltpu.sync_copy(src_hbm.at[pl.ds(offl + CHUNK, CHUNK)], sv1)
            pltpu.sync_copy(dst_hbm.at[pl.ds(offl + CHUNK, CHUNK)], dv1)
            pltpu.async_copy(g_hbm.at[sv1], rows1, gs1)
            pltpu.make_async_copy(g_hbm.at[sv0], rows0, gs0).wait()
            pltpu.async_copy(rows0, acc.at[dv0], ss0, add=True).wait()
            pltpu.make_async_copy(g_hbm.at[sv1], rows1, gs1).wait()
            pltpu.async_copy(rows1, acc.at[dv1], ss1, add=True).wait()

        @pl.when(cid == 0)
        def _():
            run(sid * (cw0 * CHUNK), cw0 // 2)

        @pl.when(cid == 1)
        def _():
            run((16 * cw0 + sid * cw1) * CHUNK, cw1 // 2)

        plsc.subcore_barrier()

        @pl.when(cid == 0)
        def _():
            pltpu.sync_copy(acc.at[pl.ds(r0, rpt)], out0_hbm.at[pl.ds(r0, rpt)])

        @pl.when(cid == 1)
        def _():
            pltpu.sync_copy(acc.at[pl.ds(r0, rpt)], out1_hbm.at[pl.ds(r0, rpt)])

    return pl.kernel(
        body,
        out_type=[jax.ShapeDtypeStruct((npad, F), jnp.float32),
                  jax.ShapeDtypeStruct((npad, F), jnp.float32)],
        mesh=mesh,
        scratch_types=(
            [pltpu.VMEM((CHUNK,), jnp.int32)] * 4
            + [pltpu.VMEM((CHUNK, F), jnp.float32)] * 2
            + [pltpu.VMEM_SHARED((npad, F), jnp.float32)]
            + [pltpu.SemaphoreType.DMA] * 4
        ),
    )


@functools.lru_cache(maxsize=None)
def _make_deg(ep, npad):
    epw = ep // NW
    nchunks = epw // CHUNK
    rpt = npad // 16
    nzfull, nzrem = divmod(rpt, CHUNK)
    mesh = plsc.VectorSubcoreMesh(core_axis_name="c", subcore_axis_name="s")

    def body(dst_hbm, out0_hbm, out1_hbm, dv0, dv1, ones_v, acc, ss0, ss1):
        cid = lax.axis_index("c")
        sid = lax.axis_index("s")
        zero16 = jnp.zeros((16,), jnp.float32)
        one16 = jnp.ones((16,), jnp.float32)

        def fill_z(i, c):
            for j in range(F // 16):
                ones_v[i, pl.ds(j * 16, 16)] = zero16
            return c
        lax.fori_loop(0, CHUNK, fill_z, 0)

        r0 = sid * rpt
        for k in range(nzfull):
            pltpu.sync_copy(ones_v, acc.at[pl.ds(r0 + k * CHUNK, CHUNK)])
        if nzrem:
            pltpu.sync_copy(ones_v.at[pl.ds(0, nzrem)],
                            acc.at[pl.ds(r0 + nzfull * CHUNK, nzrem)])

        def fill_o(i, c):
            for j in range(F // 16):
                ones_v[i, pl.ds(j * 16, 16)] = one16
            return c
        lax.fori_loop(0, CHUNK, fill_o, 0)
        plsc.subcore_barrier()

        base = (cid * 16 + sid) * epw
        nsteps = nchunks // 2

        pltpu.sync_copy(dst_hbm.at[pl.ds(base, CHUNK)], dv0)

        def step(q, c):
            off = base + q * (2 * CHUNK)
            s0 = pltpu.async_copy(ones_v, acc.at[dv0], ss0, add=True)
            pltpu.sync_copy(dst_hbm.at[pl.ds(off + CHUNK, CHUNK)], dv1)
            s1 = pltpu.async_copy(ones_v, acc.at[dv1], ss1, add=True)
            s0.wait()
            pltpu.sync_copy(dst_hbm.at[pl.ds(off + 2 * CHUNK, CHUNK)], dv0)
            s1.wait()
            return c
        lax.fori_loop(0, nsteps - 1, step, 0)

        offl = base + (nsteps - 1) * (2 * CHUNK)
        s0 = pltpu.async_copy(ones_v, acc.at[dv0], ss0, add=True)
        pltpu.sync_copy(dst_hbm.at[pl.ds(offl + CHUNK, CHUNK)], dv1)
        s1 = pltpu.async_copy(ones_v, acc.at[dv1], ss1, add=True)
        s0.wait()
        s1.wait()

        plsc.subcore_barrier()

        @pl.when(cid == 0)
        def _():
            pltpu.sync_copy(acc.at[pl.ds(r0, rpt)], out0_hbm.at[pl.ds(r0, rpt)])

        @pl.when(cid == 1)
        def _():
            pltpu.sync_copy(acc.at[pl.ds(r0, rpt)], out1_hbm.at[pl.ds(r0, rpt)])

    return pl.kernel(
        body,
        out_type=[jax.ShapeDtypeStruct((npad, F), jnp.float32),
                  jax.ShapeDtypeStruct((npad, F), jnp.float32)],
        mesh=mesh,
        scratch_types=[
            pltpu.VMEM((CHUNK,), jnp.int32),
            pltpu.VMEM((CHUNK,), jnp.int32),
            pltpu.VMEM((CHUNK, F), jnp.float32),
            pltpu.VMEM_SHARED((npad, F), jnp.float32),
            pltpu.SemaphoreType.DMA,
            pltpu.SemaphoreType.DMA,
        ],
    )


def _mm_t(x, w):
    return lax.dot_general(x, w, (((1,), (1,)), ((), ())),
                           preferred_element_type=jnp.float32)


def _pre_body(nrows, bn, x_ref, w1_ref, b1_ref, w2_ref, b2_ref,
              dp0_ref, dp1_ref, h_ref, g_ref, di_ref):
    i = pl.program_id(0)
    x = x_ref[...]
    h1 = jax.nn.relu(_mm_t(x, w1_ref[...]) + b1_ref[...])
    h2 = jax.nn.relu(_mm_t(h1, w2_ref[...]) + b2_ref[...])
    deg = dp0_ref[...][:, 0:1] + dp1_ref[...][:, 0:1]
    dinv = lax.rsqrt(jnp.maximum(deg, 1.0))
    rows = lax.broadcasted_iota(jnp.int32, (bn, 1), 0) + i * bn
    mask = (rows < nrows).astype(jnp.float32)
    h2 = h2 * mask
    h_ref[...] = h2
    g_ref[...] = h2 * dinv
    di_ref[...] = jnp.broadcast_to(dinv, (bn, 16))


def _mid_body(nrows, bn, h_ref, p0_ref, p1_ref, di_ref, c_ref, g_ref):
    i = pl.program_id(0)
    dinv = di_ref[...][:, 0:1]
    cur = h_ref[...] - (p0_ref[...] + p1_ref[...]) * dinv
    rows = lax.broadcasted_iota(jnp.int32, (bn, 1), 0) + i * bn
    mask = (rows < nrows).astype(jnp.float32)
    cur = cur * mask
    c_ref[...] = cur
    g_ref[...] = cur * dinv


def _post_body(h_ref, c1_ref, p0_ref, p1_ref, di_ref, w3_ref, b3_ref,
               w4a_ref, w4b_ref, b4_ref, hl_ref, hh_ref):
    dinv = di_ref[...][:, 0:1]
    h = h_ref[...]
    c1 = c1_ref[...]
    c2 = c1 - (p0_ref[...] + p1_ref[...]) * dinv
    o0 = _TH[0][0] * h + _TH[0][1] * c1 + _TH[0][2] * c2
    o1 = _TH[1][1] * c1 + _TH[1][2] * c2
    o2 = _TH[2][2] * c2
    hl_ref[...] = jax.nn.relu(_mm_t(o0, w3_ref[...]) + b3_ref[...])
    hh_ref[...] = jax.nn.relu(_mm_t(o1, w4a_ref[...]) + _mm_t(o2, w4b_ref[...])
                              + b4_ref[...])


def kernel(in_feat, edge_index, W1, b1, W2, b2, W3, b3, W4, b4):
    n, f = in_feat.shape
    e = edge_index.shape[1]
    npad = _round_up(n + 16, 128)
    ep = _round_up(e, NW * CHUNK * 4)

    src = edge_index[0].astype(jnp.int32)
    dst = edge_index[1].astype(jnp.int32)
    sent = jnp.full((ep - e,), n, jnp.int32)
    srcp = jnp.concatenate([src, sent])
    dstp = jnp.concatenate([dst, sent])
    xpad = jnp.pad(in_feat, ((0, npad - n), (0, 0)))
    b1r, b2r, b3r, b4r = (x.reshape(1, f) for x in (b1, b2, b3, b4))
    W4a, W4b = W4[:, :f], W4[:, f:]

    nchunks_even = ep // (NW * CHUNK)
    cw0 = _round_up(int(nchunks_even * 2 * 0.70), 2)
    prop = _make_prop(ep, npad, cw0)
    dp0, dp1 = _make_deg(ep, npad)(dstp)

    bn = npad // 4
    wspec = pl.BlockSpec((f, f), lambda i: (0, 0))
    bspec = pl.BlockSpec((1, f), lambda i: (0, 0))
    rspec = pl.BlockSpec((bn, f), lambda i: (i, 0))
    dspec = pl.BlockSpec((bn, 16), lambda i: (i, 0))
    rshape = jax.ShapeDtypeStruct((npad, f), jnp.float32)

    h, g1, dinv16 = pl.pallas_call(
        functools.partial(_pre_body, n, bn),
        grid=(npad // bn,),
        in_specs=[rspec, wspec, bspec, wspec, bspec, rspec, rspec],
        out_specs=[rspec, rspec, dspec],
        out_shape=[rshape, rshape,
                   jax.ShapeDtypeStruct((npad, 16), jnp.float32)],
    )(xpad, W1, b1r, W2, b2r, dp0, dp1)

    p10, p11 = prop(g1, srcp, dstp)

    cur1, g2 = pl.pallas_call(
        functools.partial(_mid_body, n, bn),
        grid=(npad // bn,),
        in_specs=[rspec, rspec, rspec, dspec],
        out_specs=[rspec, rspec],
        out_shape=[rshape, rshape],
    )(h, p10, p11, dinv16)

    p20, p21 = prop(g2, srcp, dstp)

    hl, hh = pl.pallas_call(
        _post_body,
        grid=(npad // bn,),
        in_specs=[rspec, rspec, rspec, rspec, dspec,
                  wspec, bspec, wspec, wspec, bspec],
        out_specs=[rspec, rspec],
        out_shape=[rshape, rshape],
    )(h, cur1, p20, p21, dinv16, W3, b3r, W4a, W4b, b4r)

    return hl[:n], hh[:n]


# speedup vs baseline: 1.4346x; 1.0060x over previous
import functools

import jax
import jax.numpy as jnp
from jax import lax
from jax.experimental import pallas as pl
from jax.experimental.pallas import tpu as pltpu
from jax.experimental.pallas import tpu_sc as plsc

F = 128
CHUNK = 128
NW = 32

_TH = ((3.0, -3.0, 0.75), (0.0, 3.0, -1.5), (0.0, 0.0, 0.75))


def _round_up(x, m):
    return ((x + m - 1) // m) * m


@functools.lru_cache(maxsize=None)
def _make_prop(ep, npad, cw0):
    nchunks = ep // (NW * CHUNK)
    cw1 = 2 * nchunks - cw0
    assert cw0 % 2 == 0 and cw1 % 2 == 0 and cw0 >= 4 and cw1 >= 4
    rpt = npad // 16
    nzfull, nzrem = divmod(rpt, CHUNK)
    mesh = plsc.VectorSubcoreMesh(core_axis_name="c", subcore_axis_name="s")

    def body(g_hbm, src_hbm, dst_hbm, out0_hbm, out1_hbm,
             sv0, sv1, dv0, dv1, rows0, rows1, acc, gs0, gs1, ss0, ss1):
        cid = lax.axis_index("c")
        sid = lax.axis_index("s")
        zero16 = jnp.zeros((16,), jnp.float32)

        def fill_z(i, c):
            for j in range(F // 16):
                rows0[i, pl.ds(j * 16, 16)] = zero16
            return c
        lax.fori_loop(0, CHUNK, fill_z, 0)

        r0 = sid * rpt
        for k in range(nzfull):
            pltpu.sync_copy(rows0, acc.at[pl.ds(r0 + k * CHUNK, CHUNK)])
        if nzrem:
            pltpu.sync_copy(rows0.at[pl.ds(0, nzrem)],
                            acc.at[pl.ds(r0 + nzfull * CHUNK, nzrem)])
        plsc.subcore_barrier()

        def run(base, nsteps):
            pltpu.sync_copy(src_hbm.at[pl.ds(base, CHUNK)], sv0)
            pltpu.sync_copy(dst_hbm.at[pl.ds(base, CHUNK)], dv0)
            pltpu.async_copy(g_hbm.at[sv0], rows0, gs0)

            def step(q, c):
                off = base + q * (2 * CHUNK)
                pltpu.sync_copy(src_hbm.at[pl.ds(off + CHUNK, CHUNK)], sv1)
                pltpu.sync_copy(dst_hbm.at[pl.ds(off + CHUNK, CHUNK)], dv1)
                pltpu.async_copy(g_hbm.at[sv1], rows1, gs1)
                pltpu.make_async_copy(g_hbm.at[sv0], rows0, gs0).wait()
                pltpu.async_copy(rows0, acc.at[dv0], ss0, add=True).wait()

                off2 = off + 2 * CHUNK
                pltpu.sync_copy(src_hbm.at[pl.ds(off2, CHUNK)], sv0)
                pltpu.sync_copy(dst_hbm.at[pl.ds(off2, CHUNK)], dv0)
                pltpu.async_copy(g_hbm.at[sv0], rows0, gs0)

                pltpu.make_async_copy(g_hbm.at[sv1], rows1, gs1).wait()
                pltpu.async_copy(rows1, acc.at[dv1], ss1, add=True).wait()
                return c
            lax.fori_loop(0, nsteps - 1, step, 0)

            offl = base + (nsteps - 1) * (2 * CHUNK)
            pltpu.sync_copy(src_hbm.at[pl.ds(offl + CHUNK, CHUNK)], sv1)
            pltpu.sync_copy(dst_hbm.at[pl.ds(offl + CHUNK, CHUNK)], dv1)
            pltpu.async_copy(g_hbm.at[sv1], rows1, gs1)
            pltpu.make_async_copy(g_hbm.at[sv0], rows0, gs0).wait()
            pltpu.async_copy(rows0, acc.at[dv0], ss0, add=True).wait()
            pltpu.make_async_copy(g_hbm.at[sv1], rows1, gs1).wait()
            pltpu.async_copy(rows1, acc.at[dv1], ss1, add=True).wait()

        @pl.when(cid == 0)
        def _():
            run(sid * (cw0 * CHUNK), cw0 // 2)

        @pl.when(cid == 1)
        def _():
            run((16 * cw0 + sid * cw1) * CHUNK, cw1 // 2)

        plsc.subcore_barrier()

        @pl.when(cid == 0)
        def _():
            pltpu.sync_copy(acc.at[pl.ds(r0, rpt)], out0_hbm.at[pl.ds(r0, rpt)])

        @pl.when(cid == 1)
        def _():
            pltpu.sync_copy(acc.at[pl.ds(r0, rpt)], out1_hbm.at[pl.ds(r0, rpt)])

    return pl.kernel(
        body,
        out_type=[jax.ShapeDtypeStruct((npad, F), jnp.float32),
                  jax.ShapeDtypeStruct((npad, F), jnp.float32)],
        mesh=mesh,
        scratch_types=(
            [pltpu.VMEM((CHUNK,), jnp.int32)] * 4
            + [pltpu.VMEM((CHUNK, F), jnp.float32)] * 2
            + [pltpu.VMEM_SHARED((npad, F), jnp.float32)]
            + [pltpu.SemaphoreType.DMA] * 4
        ),
    )


@functools.lru_cache(maxsize=None)
def _make_deg(ep, npad):
    epw = ep // NW
    nchunks = epw // CHUNK
    rpt = npad // 16
    nzfull, nzrem = divmod(rpt, CHUNK)
    mesh = plsc.VectorSubcoreMesh(core_axis_name="c", subcore_axis_name="s")

    def body(dst_hbm, out0_hbm, out1_hbm, dv0, dv1, ones_v, acc, ss0, ss1):
        cid = lax.axis_index("c")
        sid = lax.axis_index("s")
        zero16 = jnp.zeros((16,), jnp.float32)
        one16 = jnp.ones((16,), jnp.float32)

        def fill_z(i, c):
            for j in range(F // 16):
                ones_v[i, pl.ds(j * 16, 16)] = zero16
            return c
        lax.fori_loop(0, CHUNK, fill_z, 0)

        r0 = sid * rpt
        for k in range(nzfull):
            pltpu.sync_copy(ones_v, acc.at[pl.ds(r0 + k * CHUNK, CHUNK)])
        if nzrem:
            pltpu.sync_copy(ones_v.at[pl.ds(0, nzrem)],
                            acc.at[pl.ds(r0 + nzfull * CHUNK, nzrem)])

        def fill_o(i, c):
            for j in range(F // 16):
                ones_v[i, pl.ds(j * 16, 16)] = one16
            return c
        lax.fori_loop(0, CHUNK, fill_o, 0)
        plsc.subcore_barrier()

        base = (cid * 16 + sid) * epw
        nsteps = nchunks // 2

        pltpu.sync_copy(dst_hbm.at[pl.ds(base, CHUNK)], dv0)

        def step(q, c):
            off = base + q * (2 * CHUNK)
            s0 = pltpu.async_copy(ones_v, acc.at[dv0], ss0, add=True)
            pltpu.sync_copy(dst_hbm.at[pl.ds(off + CHUNK, CHUNK)], dv1)
            s1 = pltpu.async_copy(ones_v, acc.at[dv1], ss1, add=True)
            s0.wait()
            pltpu.sync_copy(dst_hbm.at[pl.ds(off + 2 * CHUNK, CHUNK)], dv0)
            s1.wait()
            return c
        lax.fori_loop(0, nsteps - 1, step, 0)

        offl = base + (nsteps - 1) * (2 * CHUNK)
        s0 = pltpu.async_copy(ones_v, acc.at[dv0], ss0, add=True)
        pltpu.sync_copy(dst_hbm.at[pl.ds(offl + CHUNK, CHUNK)], dv1)
        s1 = pltpu.async_copy(ones_v, acc.at[dv1], ss1, add=True)
        s0.wait()
        s1.wait()

        plsc.subcore_barrier()

        @pl.when(cid == 0)
        def _():
            pltpu.sync_copy(acc.at[pl.ds(r0, rpt)], out0_hbm.at[pl.ds(r0, rpt)])

        @pl.when(cid == 1)
        def _():
            pltpu.sync_copy(acc.at[pl.ds(r0, rpt)], out1_hbm.at[pl.ds(r0, rpt)])

    return pl.kernel(
        body,
        out_type=[jax.ShapeDtypeStruct((npad, F), jnp.float32),
                  jax.ShapeDtypeStruct((npad, F), jnp.float32)],
        mesh=mesh,
        scratch_types=[
            pltpu.VMEM((CHUNK,), jnp.int32),
            pltpu.VMEM((CHUNK,), jnp.int32),
            pltpu.VMEM((CHUNK, F), jnp.float32),
            pltpu.VMEM_SHARED((npad, F), jnp.float32),
            pltpu.SemaphoreType.DMA,
            pltpu.SemaphoreType.DMA,
        ],
    )


def _mm_t(x, w):
    return lax.dot_general(x, w, (((1,), (1,)), ((), ())),
                           preferred_element_type=jnp.float32)


def _pre_body(nrows, bn, x_ref, w1_ref, b1_ref, w2_ref, b2_ref,
              dp0_ref, dp1_ref, h_ref, g_ref, di_ref):
    i = pl.program_id(0)
    x = x_ref[...]
    h1 = jax.nn.relu(_mm_t(x, w1_ref[...]) + b1_ref[...])
    h2 = jax.nn.relu(_mm_t(h1, w2_ref[...]) + b2_ref[...])
    deg = dp0_ref[...][:, 0:1] + dp1_ref[...][:, 0:1]
    dinv = lax.rsqrt(jnp.maximum(deg, 1.0))
    rows = lax.broadcasted_iota(jnp.int32, (bn, 1), 0) + i * bn
    mask = (rows < nrows).astype(jnp.float32)
    h2 = h2 * mask
    h_ref[...] = h2
    g_ref[...] = h2 * dinv
    di_ref[...] = jnp.broadcast_to(dinv, (bn, 16))


def _mid_body(nrows, bn, h_ref, p0_ref, p1_ref, di_ref, c_ref, g_ref):
    i = pl.program_id(0)
    dinv = di_ref[...][:, 0:1]
    cur = h_ref[...] - (p0_ref[...] + p1_ref[...]) * dinv
    rows = lax.broadcasted_iota(jnp.int32, (bn, 1), 0) + i * bn
    mask = (rows < nrows).astype(jnp.float32)
    cur = cur * mask
    c_ref[...] = cur
    g_ref[...] = cur * dinv


def _post_body(h_ref, c1_ref, p0_ref, p1_ref, di_ref, w3_ref, b3_ref,
               w4a_ref, w4b_ref, b4_ref, hl_ref, hh_ref):
    dinv = di_ref[...][:, 0:1]
    h = h_ref[...]
    c1 = c1_ref[...]
    c2 = c1 - (p0_ref[...] + p1_ref[...]) * dinv
    o0 = _TH[0][0] * h + _TH[0][1] * c1 + _TH[0][2] * c2
    o1 = _TH[1][1] * c1 + _TH[1][2] * c2
    o2 = _TH[2][2] * c2
    hl_ref[...] = jax.nn.relu(_mm_t(o0, w3_ref[...]) + b3_ref[...])
    hh_ref[...] = jax.nn.relu(_mm_t(o1, w4a_ref[...]) + _mm_t(o2, w4b_ref[...])
                              + b4_ref[...])


def kernel(in_feat, edge_index, W1, b1, W2, b2, W3, b3, W4, b4):
    n, f = in_feat.shape
    e = edge_index.shape[1]
    npad = _round_up(n + 16, 128)
    ep = _round_up(e, NW * CHUNK * 4)

    src = edge_index[0].astype(jnp.int32)
    dst = edge_index[1].astype(jnp.int32)
    sent = jnp.full((ep - e,), n, jnp.int32)
    srcp = jnp.concatenate([src, sent])
    dstp = jnp.concatenate([dst, sent])
    xpad = jnp.pad(in_feat, ((0, npad - n), (0, 0)))
    b1r, b2r, b3r, b4r = (x.reshape(1, f) for x in (b1, b2, b3, b4))
    W4a, W4b = W4[:, :f], W4[:, f:]

    nchunks_even = ep // (NW * CHUNK)
    cw0 = _round_up(int(nchunks_even * 2 * 0.75), 2)
    prop = _make_prop(ep, npad, cw0)
    dp0, dp1 = _make_deg(ep, npad)(dstp)

    bn = npad // 4
    wspec = pl.BlockSpec((f, f), lambda i: (0, 0))
    bspec = pl.BlockSpec((1, f), lambda i: (0, 0))
    rspec = pl.BlockSpec((bn, f), lambda i: (i, 0))
    dspec = pl.BlockSpec((bn, 16), lambda i: (i, 0))
    rshape = jax.ShapeDtypeStruct((npad, f), jnp.float32)

    h, g1, dinv16 = pl.pallas_call(
        functools.partial(_pre_body, n, bn),
        grid=(npad // bn,),
        in_specs=[rspec, wspec, bspec, wspec, bspec, rspec, rspec],
        out_specs=[rspec, rspec, dspec],
        out_shape=[rshape, rshape,
                   jax.ShapeDtypeStruct((npad, 16), jnp.float32)],
    )(xpad, W1, b1r, W2, b2r, dp0, dp1)

    p10, p11 = prop(g1, srcp, dstp)

    cur1, g2 = pl.pallas_call(
        functools.partial(_mid_body, n, bn),
        grid=(npad // bn,),
        in_specs=[rspec, rspec, rspec, dspec],
        out_specs=[rspec, rspec],
        out_shape=[rshape, rshape],
    )(h, p10, p11, dinv16)

    p20, p21 = prop(g2, srcp, dstp)

    hl, hh = pl.pallas_call(
        _post_body,
        grid=(npad // bn,),
        in_specs=[rspec, rspec, rspec, rspec, dspec,
                  wspec, bspec, wspec, wspec, bspec],
        out_specs=[rspec, rspec],
        out_shape=[rshape, rshape],
    )(h, cur1, p20, p21, dinv16, W3, b3r, W4a, W4b, b4r)

    return hl[:n], hh[:n]
